# baseline (speedup 1.0000x reference)
import functools

import jax
import jax.numpy as jnp
import numpy as np
from jax.experimental import pallas as pl
from jax.experimental.pallas import tpu as pltpu

_NTYPE = 4
_NWAVE = 8
_NIPSIN = 3
_NANG = 13
_NORBIT = 32
_CUTOFF = 5.0
_AP = 1024
_P = 1280

_INDEX_PARA = (0, 1, 1, 1, 2, 2, 2, 2, 2, 2, 2, 2, 2)


def _mm(a, b, dims):
    return jax.lax.dot_general(a, b, (dims, ((), ())),
                               preferred_element_type=jnp.float32)


def _body(cart_hi_ref, cart_lo_ref, idx_ref, shifts_ref, sp_ref, ef_ref,
          rs_ref, inta_ref, par_ref, efp_ref, hyp_ref, out_ref, acc_ref,
          tabh_ref, tabl_ref):
    j = pl.program_id(1)
    nblk = pl.num_programs(1)

    @pl.when(j == 0)
    def _init():
        acc_ref[...] = jnp.zeros_like(acc_ref)
        sp = sp_ref[0]
        sp_oh = (jax.lax.broadcasted_iota(jnp.int32, (_NTYPE, _AP), 0)
                 == sp).astype(jnp.float32)
        tabs = jnp.concatenate([rs_ref[...], inta_ref[...], par_ref[...]],
                               axis=0)
        tab = _mm(tabs, sp_oh, ((1,), (0,)))
        th = tab.astype(jnp.bfloat16)
        tabh_ref[...] = th
        tabl_ref[...] = (tab - th.astype(jnp.float32)).astype(jnp.bfloat16)

    ids = idx_ref[0]
    idx0 = ids[0:1]
    idx1 = ids[1:2]
    iota_a = jax.lax.broadcasted_iota(jnp.int32, (_AP, _P), 0)
    oh0 = (iota_a == idx0).astype(jnp.bfloat16)
    oh1 = (iota_a == idx1).astype(jnp.bfloat16)
    ohd = oh1 - oh0

    d_raw = (_mm(cart_hi_ref[0], ohd, ((1,), (0,)))
             + _mm(cart_lo_ref[0], ohd, ((1,), (0,))))
    g = (_mm(tabh_ref[...], oh1, ((1,), (0,)))
         + _mm(tabl_ref[...], oh1, ((1,), (0,))))

    dvec = d_raw[0:3] + shifts_ref[0]
    d2 = jnp.sum(dvec * dvec, axis=0, keepdims=True)
    d = jnp.sqrt(d2)
    inv_d = 1.0 / d
    c = 0.5 * jnp.cos(d * (np.pi / _CUTOFF)) + 0.5
    dcut = c * c
    rs_a, inta_a, par_a = g[0:8], g[8:16], g[16:24]
    dr = d - rs_a
    rw = jnp.exp(inta_a * dr * dr) * par_a
    u = dvec * inv_d

    angs = [dcut]
    for a in range(3):
        angs.append(dcut * u[a:a + 1])
    for a in range(3):
        for b in range(3):
            angs.append(angs[1 + a] * u[b:b + 1])
    w_t = jnp.concatenate([rw * ang for ang in angs], axis=0)
    w_hi = w_t.astype(jnp.bfloat16)
    w_lo = (w_t - w_hi.astype(jnp.float32)).astype(jnp.bfloat16)
    acc_ref[...] += (_mm(w_hi, oh0, ((1,), (1,)))
                     + _mm(w_lo, oh0, ((1,), (1,))))

    @pl.when(j == nblk - 1)
    def _finish():
        e = [ef_ref[0, 0, 0], ef_ref[0, 0, 1], ef_ref[0, 0, 2]]
        ef_ang = [1.0] + e + [e[a] * e[b] for a in range(3) for b in range(3)]
        efp = efp_ref[...]
        dens = jnp.zeros((_AP, _NORBIT), jnp.float32)
        for jj in range(_NANG):
            eo = acc_ref[jj * 8:(jj + 1) * 8, :] + efp * ef_ang[jj]
            h = hyp_ref[_INDEX_PARA[jj]]
            hw = jax.lax.dot_general(eo, h, (((0,), (0,)), ((), ())),
                                     preferred_element_type=jnp.float32)
            dens = dens + hw * hw
        out_ref[...] = dens[:1000, :]


@jax.jit
def kernel(cart, ef, numatoms, species, atom_index, shifts, rs, inta, params,
           ef_para, hyper):
    del numatoms
    nbatch, numatom, _ = cart.shape
    npair = atom_index.shape[2]
    nblk = npair // _P

    cart_t = jnp.zeros((nbatch, 8, _AP), jnp.float32)
    cart_t = cart_t.at[:, 0:3, :numatom].set(cart.transpose(0, 2, 1))
    cart_hi = cart_t.astype(jnp.bfloat16)
    cart_lo = (cart_t - cart_hi.astype(jnp.float32)).astype(jnp.bfloat16)
    idx_t = atom_index.transpose(1, 0, 2).astype(jnp.int32)
    shifts_t = shifts.transpose(0, 2, 1)
    sp_p = jnp.zeros((nbatch, 1, _AP), jnp.int32)
    sp_p = sp_p.at[:, 0, :numatom].set(
        species.reshape(nbatch, numatom).astype(jnp.int32))
    ef_r = ef.reshape(nbatch, 1, 3)
    rs_t, inta_t, par_t = rs.T, inta.T, params.T
    efp_c = ef_para.reshape(_NWAVE, 1)

    grid = (nbatch, nblk)
    out = pl.pallas_call(
        _body,
        grid=grid,
        in_specs=[
            pl.BlockSpec((1, 8, _AP), lambda b, j: (b, 0, 0)),
            pl.BlockSpec((1, 8, _AP), lambda b, j: (b, 0, 0)),
            pl.BlockSpec((1, 2, _P), lambda b, j: (b, 0, j)),
            pl.BlockSpec((1, 3, _P), lambda b, j: (b, 0, j)),
            pl.BlockSpec((1, 1, _AP), lambda b, j: (b, 0, 0)),
            pl.BlockSpec((1, 1, 3), lambda b, j: (b, 0, 0),
                         memory_space=pltpu.SMEM),
            pl.BlockSpec((8, 4), lambda b, j: (0, 0)),
            pl.BlockSpec((8, 4), lambda b, j: (0, 0)),
            pl.BlockSpec((8, 4), lambda b, j: (0, 0)),
            pl.BlockSpec((8, 1), lambda b, j: (0, 0)),
            pl.BlockSpec((3, 8, 32), lambda b, j: (0, 0, 0)),
        ],
        out_specs=pl.BlockSpec((numatom, _NORBIT), lambda b, j: (b, 0)),
        out_shape=jax.ShapeDtypeStruct((nbatch * numatom, _NORBIT),
                                       jnp.float32),
        scratch_shapes=[
            pltpu.VMEM((_NANG * _NWAVE, _AP), jnp.float32),
            pltpu.VMEM((24, _AP), jnp.bfloat16),
            pltpu.VMEM((24, _AP), jnp.bfloat16),
        ],
        compiler_params=pltpu.CompilerParams(
            dimension_semantics=("arbitrary", "arbitrary")),
    )(cart_hi, cart_lo, idx_t, shifts_t, sp_p, ef_r, rs_t, inta_t, par_t,
      efp_c, hyper)
    return out


# speedup vs baseline: 1.7357x; 1.7357x over previous
import functools

import jax
import jax.numpy as jnp
import numpy as np
from jax.experimental import pallas as pl
from jax.experimental.pallas import tpu as pltpu

_NTYPE = 4
_NWAVE = 8
_NIPSIN = 3
_NANG = 13
_NORBIT = 32
_CUTOFF = 5.0
_AP = 1024
_P = 1280

_INDEX_PARA = (0, 1, 1, 1, 2, 2, 2, 2, 2, 2, 2, 2, 2)


def _mm(a, b, dims):
    return jax.lax.dot_general(a, b, (dims, ((), ())),
                               preferred_element_type=jnp.float32)


def _body(cart_ref, idx_ref, shifts_ref, sp_ref, ef_ref,
          rs_ref, inta_ref, par_ref, efp_ref, hyp_ref, out_ref, acc_ref,
          tab_ref):
    j = pl.program_id(1)
    nblk = pl.num_programs(1)

    @pl.when(j == 0)
    def _init():
        acc_ref[...] = jnp.zeros_like(acc_ref)
        sp = sp_ref[0]
        sp_oh = (jax.lax.broadcasted_iota(jnp.int32, (_NTYPE, _AP), 0)
                 == sp).astype(jnp.float32)
        tabs = jnp.concatenate([rs_ref[...], inta_ref[...], par_ref[...]],
                               axis=0)
        tab_ref[...] = _mm(tabs, sp_oh, ((1,), (0,)))

    ids = idx_ref[0]
    idx0 = ids[0:1]
    idx1 = ids[1:2]
    iota_a = jax.lax.broadcasted_iota(jnp.int32, (_AP, _P), 0)
    oh0 = (iota_a == idx0).astype(jnp.float32)
    oh1 = (iota_a == idx1).astype(jnp.float32)

    d_raw = _mm(cart_ref[0], oh1 - oh0, ((1,), (0,)))
    g = _mm(tab_ref[...], oh1, ((1,), (0,)))

    dvec = d_raw[0:3] + shifts_ref[0]
    d2 = jnp.sum(dvec * dvec, axis=0, keepdims=True)
    d = jnp.sqrt(d2)
    inv_d = 1.0 / d
    c = 0.5 * jnp.cos(d * (np.pi / _CUTOFF)) + 0.5
    dcut = c * c
    rs_a, inta_a, par_a = g[0:8], g[8:16], g[16:24]
    dr = d - rs_a
    rw = jnp.exp(inta_a * dr * dr) * par_a
    u = dvec * inv_d

    angs = [dcut]
    for a in range(3):
        angs.append(dcut * u[a:a + 1])
    for a in range(3):
        for b in range(3):
            angs.append(angs[1 + a] * u[b:b + 1])
    w_t = jnp.concatenate([rw * ang for ang in angs], axis=0)
    acc_ref[...] += _mm(w_t, oh0, ((1,), (1,)))

    @pl.when(j == nblk - 1)
    def _finish():
        e = [ef_ref[0, 0, 0], ef_ref[0, 0, 1], ef_ref[0, 0, 2]]
        ef_ang = [1.0] + e + [e[a] * e[b] for a in range(3) for b in range(3)]
        efp = efp_ref[...]
        dens = jnp.zeros((_AP, _NORBIT), jnp.float32)
        for jj in range(_NANG):
            eo = acc_ref[jj * 8:(jj + 1) * 8, :] + efp * ef_ang[jj]
            h = hyp_ref[_INDEX_PARA[jj]]
            hw = jax.lax.dot_general(eo, h, (((0,), (0,)), ((), ())),
                                     preferred_element_type=jnp.float32)
            dens = dens + hw * hw
        out_ref[...] = dens[:1000, :]


@jax.jit
def kernel(cart, ef, numatoms, species, atom_index, shifts, rs, inta, params,
           ef_para, hyper):
    del numatoms
    nbatch, numatom, _ = cart.shape
    npair = atom_index.shape[2]
    nblk = npair // _P

    cart_t = jnp.zeros((nbatch, 8, _AP), jnp.float32)
    cart_t = cart_t.at[:, 0:3, :numatom].set(cart.transpose(0, 2, 1))
    idx_t = atom_index.transpose(1, 0, 2).astype(jnp.int32)
    shifts_t = shifts.transpose(0, 2, 1)
    sp_p = jnp.zeros((nbatch, 1, _AP), jnp.int32)
    sp_p = sp_p.at[:, 0, :numatom].set(
        species.reshape(nbatch, numatom).astype(jnp.int32))
    ef_r = ef.reshape(nbatch, 1, 3)
    rs_t, inta_t, par_t = rs.T, inta.T, params.T
    efp_c = ef_para.reshape(_NWAVE, 1)

    grid = (nbatch, nblk)
    out = pl.pallas_call(
        _body,
        grid=grid,
        in_specs=[
            pl.BlockSpec((1, 8, _AP), lambda b, j: (b, 0, 0)),
            pl.BlockSpec((1, 2, _P), lambda b, j: (b, 0, j)),
            pl.BlockSpec((1, 3, _P), lambda b, j: (b, 0, j)),
            pl.BlockSpec((1, 1, _AP), lambda b, j: (b, 0, 0)),
            pl.BlockSpec((1, 1, 3), lambda b, j: (b, 0, 0),
                         memory_space=pltpu.SMEM),
            pl.BlockSpec((8, 4), lambda b, j: (0, 0)),
            pl.BlockSpec((8, 4), lambda b, j: (0, 0)),
            pl.BlockSpec((8, 4), lambda b, j: (0, 0)),
            pl.BlockSpec((8, 1), lambda b, j: (0, 0)),
            pl.BlockSpec((3, 8, 32), lambda b, j: (0, 0, 0)),
        ],
        out_specs=pl.BlockSpec((numatom, _NORBIT), lambda b, j: (b, 0)),
        out_shape=jax.ShapeDtypeStruct((nbatch * numatom, _NORBIT),
                                       jnp.float32),
        scratch_shapes=[
            pltpu.VMEM((_NANG * _NWAVE, _AP), jnp.float32),
            pltpu.VMEM((24, _AP), jnp.float32),
        ],
        compiler_params=pltpu.CompilerParams(
            dimension_semantics=("arbitrary", "arbitrary")),
    )(cart_t, idx_t, shifts_t, sp_p, ef_r, rs_t, inta_t, par_t,
      efp_c, hyper)
    return out


# speedup vs baseline: 2.6232x; 1.5113x over previous
import functools

import jax
import jax.numpy as jnp
import numpy as np
from jax import lax
from jax.experimental import pallas as pl
from jax.experimental.pallas import tpu as pltpu
from jax.experimental.pallas import tpu_sc as plsc

_NTYPE = 4
_NWAVE = 8
_NANG = 13
_NORBIT = 32
_CUTOFF = 5.0
_NB = 10
_NA = 1000
_NP = 32000
_NC, _NS, _L = 2, 16, 16
_NW = _NC * _NS

_INDEX_PARA = (0, 1, 1, 1, 2, 2, 2, 2, 2, 2, 2, 2, 2)

_UNIT = 1280
_UNITS = (_NB * _NP) // _UNIT
_UNITS_PER_BATCH = _NP // _UNIT
_ROUNDS = (_UNITS + _NW - 1) // _NW

_PART = 10752
_CCH = 256
_ACC = _NANG * _NWAVE * 1024

_MESH = plsc.VectorSubcoreMesh(core_axis_name="c", subcore_axis_name="s")


def _wid():
    return lax.axis_index("s") * _NC + lax.axis_index("c")


def _sc_gather_body(cart_hbm, spec_hbm, i0_hbm, i1_hbm, outf_hbm,
                    cart_v, spec_v, idx0_v, idx1_v, stage_v):
    wid = _wid()
    for i in range(_ROUNDS):
        u = wid + i * _NW

        @pl.when(u < _UNITS)
        def _unit():
            b = u // _UNITS_PER_BATCH
            off = (u % _UNITS_PER_BATCH) * _UNIT
            pltpu.sync_copy(cart_hbm.at[pl.ds(b * 3 * _NA, 3 * _NA)], cart_v)
            pltpu.sync_copy(spec_hbm.at[pl.ds(b * _NA, _NA)], spec_v)
            pltpu.sync_copy(i0_hbm.at[pl.ds(b * _NP + off, _UNIT)], idx0_v)
            pltpu.sync_copy(i1_hbm.at[pl.ds(b * _NP + off, _UNIT)], idx1_v)

            def grp(g, carry):
                i0 = idx0_v[pl.ds(g * _L, _L)]
                i1 = idx1_v[pl.ds(g * _L, _L)]
                s = plsc.load_gather(spec_v, [i1])
                a0 = i0 * 3
                a1 = i1 * 3
                for c in range(3):
                    c0 = plsc.load_gather(cart_v, [a0 + c])
                    c1 = plsc.load_gather(cart_v, [a1 + c])
                    stage_v[c, pl.ds(g * _L, _L)] = c1 - c0
                stage_v[3, pl.ds(g * _L, _L)] = s.astype(jnp.float32)
                return carry

            lax.fori_loop(0, _UNIT // _L, grp, 0)
            pltpu.sync_copy(stage_v,
                            outf_hbm.at[:, pl.ds(b * _NP + off, _UNIT)])


_PB = 6400


def _tc_pair_body(f_ref, sh_ref, rs_ref, inta_ref, par_ref, out_ref):
    f = f_ref[...]
    dvec = f[0:3] + sh_ref[...]
    s = f[3:4]
    d2 = jnp.sum(dvec * dvec, axis=0, keepdims=True)
    d = jnp.sqrt(d2)
    inv_d = 1.0 / d
    c = 0.5 * jnp.cos(d * (np.pi / _CUTOFF)) + 0.5
    dcut = c * c

    rs_a = jnp.zeros((_NWAVE, _PB), jnp.float32)
    inta_a = jnp.zeros((_NWAVE, _PB), jnp.float32)
    par_a = jnp.zeros((_NWAVE, _PB), jnp.float32)
    for t in range(_NTYPE):
        m = s == float(t)
        rs_a = jnp.where(m, rs_ref[:, t:t + 1], rs_a)
        inta_a = jnp.where(m, inta_ref[:, t:t + 1], inta_a)
        par_a = jnp.where(m, par_ref[:, t:t + 1], par_a)

    dr = d - rs_a
    rw = jnp.exp(inta_a * dr * dr) * par_a
    u = dvec * inv_d
    angs = [dcut]
    for a in range(3):
        angs.append(dcut * u[a:a + 1])
    for a in range(3):
        for b in range(3):
            angs.append(angs[1 + a] * u[b:b + 1])
    out_ref[...] = jnp.concatenate(
        angs + [rw, jnp.zeros((3, _PB), jnp.float32)], axis=0)


def _sc_scatter_body(pairf_hbm, i0_hbm, zeros_hbm, outc_hbm,
                     acc_v, fchunk_v, ichunk_v):
    wid = _wid()

    @pl.when(wid < 3 * _NB)
    def _():
        b = wid // 3
        q = wid % 3
        part_off = q * _PART
        nch = jnp.where(q == 2, (_NP - 2 * _PART) // _CCH, _PART // _CCH)
        pltpu.sync_copy(zeros_hbm, acc_v)

        def chunk(ci, carry):
            col = part_off + ci * _CCH
            pltpu.sync_copy(pairf_hbm.at[:, pl.ds(b * _NP + col, _CCH)],
                            fchunk_v)
            pltpu.sync_copy(i0_hbm.at[pl.ds(b * _NP + col, _CCH)], ichunk_v)

            def grp(g, carry2):
                i0 = ichunk_v[pl.ds(g * _L, _L)]
                rws = [fchunk_v[_NANG + k, pl.ds(g * _L, _L)]
                       for k in range(_NWAVE)]
                for j in range(_NANG):
                    aj = fchunk_v[j, pl.ds(g * _L, _L)]
                    for k in range(_NWAVE):
                        plsc.addupdate_scatter(
                            acc_v, [i0 + (j * _NWAVE + k) * 1024],
                            aj * rws[k])
                return carry2

            lax.fori_loop(0, _CCH // _L, grp, 0)
            return carry

        lax.fori_loop(0, nch, chunk, 0)
        pltpu.sync_copy(acc_v, outc_hbm.at[pl.ds(wid * _ACC, _ACC)])


def _tc_contract_body(p_ref, ef_ref, efp_ref, hyp_ref, out_ref):
    p = p_ref[0]
    eot = p[0] + p[1] + p[2]
    e = [ef_ref[0, 0, 0], ef_ref[0, 0, 1], ef_ref[0, 0, 2]]
    ef_ang = [1.0] + e + [e[a] * e[b] for a in range(3) for b in range(3)]
    base = jnp.concatenate([efp_ref[...] * ef_ang[j] for j in range(_NANG)],
                           axis=0)
    eot = eot + base
    dens = jnp.zeros((_NORBIT, 1024), jnp.float32)
    for jj in range(_NANG):
        h = hyp_ref[_INDEX_PARA[jj]]
        hw = jax.lax.dot_general(h, eot[jj * 8:(jj + 1) * 8],
                                 (((0,), (0,)), ((), ())),
                                 preferred_element_type=jnp.float32)
        dens = dens + hw * hw
    out_ref[...] = jnp.transpose(dens)[:_NA, :]


@jax.jit
def kernel(cart, ef, numatoms, species, atom_index, shifts, rs, inta, params,
           ef_para, hyper):
    del numatoms
    cart2 = cart.reshape(_NB * 3 * _NA).astype(jnp.float32)
    spec2 = species.astype(jnp.int32)
    ai = atom_index.astype(jnp.int32)
    i0_flat = ai[0].reshape(_NB * _NP)
    i1_flat = ai[1].reshape(_NB * _NP)
    shifts_f = shifts.transpose(2, 0, 1).reshape(3, _NB * _NP)
    rs_t, inta_t, par_t = rs.T, inta.T, params.T
    ef_r = ef.reshape(_NB, 1, 3)
    efp_c = ef_para.reshape(_NWAVE, 1)

    sc_gather = functools.partial(
        pl.kernel,
        out_type=jax.ShapeDtypeStruct((4, _NB * _NP), jnp.float32),
        mesh=_MESH,
        scratch_types=[
            pltpu.VMEM((3 * _NA,), jnp.float32),
            pltpu.VMEM((_NA,), jnp.int32),
            pltpu.VMEM((_UNIT,), jnp.int32),
            pltpu.VMEM((_UNIT,), jnp.int32),
            pltpu.VMEM((4, _UNIT), jnp.float32),
        ],
        compiler_params=pltpu.CompilerParams(needs_layout_passes=False),
    )(_sc_gather_body)
    outf = sc_gather(cart2, spec2, i0_flat, i1_flat)

    pairf = pl.pallas_call(
        _tc_pair_body,
        grid=(_NB * _NP // _PB,),
        in_specs=[
            pl.BlockSpec((4, _PB), lambda i: (0, i)),
            pl.BlockSpec((3, _PB), lambda i: (0, i)),
            pl.BlockSpec((8, 4), lambda i: (0, 0)),
            pl.BlockSpec((8, 4), lambda i: (0, 0)),
            pl.BlockSpec((8, 4), lambda i: (0, 0)),
        ],
        out_specs=pl.BlockSpec((24, _PB), lambda i: (0, i)),
        out_shape=jax.ShapeDtypeStruct((24, _NB * _NP), jnp.float32),
    )(outf, shifts_f, rs_t, inta_t, par_t)

    sc_scatter = functools.partial(
        pl.kernel,
        out_type=jax.ShapeDtypeStruct((3 * _NB * _ACC,), jnp.float32),
        mesh=_MESH,
        scratch_types=[
            pltpu.VMEM((_ACC,), jnp.float32),
            pltpu.VMEM((24, _CCH), jnp.float32),
            pltpu.VMEM((_CCH,), jnp.int32),
        ],
        compiler_params=pltpu.CompilerParams(needs_layout_passes=False),
    )(_sc_scatter_body)
    outc = sc_scatter(pairf, i0_flat, jnp.zeros((_ACC,), jnp.float32))

    parts = outc.reshape(_NB, 3, _NANG * _NWAVE, 1024)
    out = pl.pallas_call(
        _tc_contract_body,
        grid=(_NB,),
        in_specs=[
            pl.BlockSpec((1, 3, _NANG * _NWAVE, 1024), lambda b: (b, 0, 0, 0)),
            pl.BlockSpec((1, 1, 3), lambda b: (b, 0, 0),
                         memory_space=pltpu.SMEM),
            pl.BlockSpec((8, 1), lambda b: (0, 0)),
            pl.BlockSpec((3, 8, 32), lambda b: (0, 0, 0)),
        ],
        out_specs=pl.BlockSpec((_NA, _NORBIT), lambda b: (b, 0)),
        out_shape=jax.ShapeDtypeStruct((_NB * _NA, _NORBIT), jnp.float32),
    )(parts, ef_r, efp_c, hyper)
    return out
